# Initial kernel scaffold; baseline (speedup 1.0000x reference)
import jax
import jax.numpy as jnp
from jax.experimental import pallas as pl


def kernel(x, edge_index, W, att_src, att_dst, bias, bn_gamma, bn_beta, bn_mean, bn_var):
    raise NotImplementedError("write your pallas kernel here")


# speedup vs baseline: 26.0463x; 26.0463x over previous
import functools

import jax
import jax.numpy as jnp
from jax import lax
from jax.experimental import pallas as pl
from jax.experimental.pallas import tpu as pltpu
from jax.experimental.pallas import tpu_sc as plsc

N = 10000
NP = 10240
H = 128
E_RAW = 320000
E_SELF = E_RAW + N
NC = 2
NS = 16
NW = NC * NS
C = 128
K = -(-E_SELF // (NW * C))
E_PAD = NW * C * K
ROWS_PER_TILE = NP // NS
_ZCHUNKS = (128,) * (ROWS_PER_TILE // 128)


def _proj_body(x_ref, w_ref, asv_ref, adv_ref, h_ref, as_ref, ad_ref, sh_ref):
    h = jnp.dot(x_ref[...], w_ref[...], preferred_element_type=jnp.float32)
    h_ref[...] = h
    a_s = jnp.sum(h * asv_ref[...], axis=1, keepdims=True)
    a_d = jnp.sum(h * adv_ref[...], axis=1, keepdims=True)
    valid = jax.lax.broadcasted_iota(jnp.int32, (NP, 1), 0) < N
    a_s = jnp.where(valid, a_s, -1e30)
    a_d = jnp.where(valid, a_d, -1e30)
    as_ref[...] = a_s
    ad_ref[...] = a_d
    shift = jnp.max(a_s) + jnp.max(a_d)
    sh_ref[...] = jnp.full((1, 16), shift, jnp.float32)


def _project(x_pad, W, att_src, att_dst):
    return pl.pallas_call(
        _proj_body,
        out_shape=[
            jax.ShapeDtypeStruct((NP, H), jnp.float32),
            jax.ShapeDtypeStruct((NP, 1), jnp.float32),
            jax.ShapeDtypeStruct((NP, 1), jnp.float32),
            jax.ShapeDtypeStruct((1, 16), jnp.float32),
        ],
    )(x_pad, W, att_src.reshape(1, H), att_dst.reshape(1, H))


def _edge_body(h_hbm, as_hbm, ad_hbm, sh_hbm, src_hbm, dst_hbm,
               out_hbm, den_hbm,
               asrc_v, adst_v, shift_v, srcb, dstb, wbuf, rows, out_sp, den_sp):
    cid = lax.axis_index("c")
    sid = lax.axis_index("s")
    worker = cid * NS + sid

    pltpu.sync_copy(as_hbm, asrc_v)
    pltpu.sync_copy(ad_hbm, adst_v)
    pltpu.sync_copy(sh_hbm, shift_v)
    shift_vec = shift_v[...]

    zf = jnp.zeros((16,), jnp.float32)

    @pl.loop(0, C)
    def _(r):
        for t in range(H // 16):
            rows[r, pl.ds(16 * t, 16)] = zf

    for t in range(C // 16):
        wbuf[pl.ds(16 * t, 16)] = zf

    row0 = sid * ROWS_PER_TILE
    off = 0
    for sz in _ZCHUNKS:
        pltpu.sync_copy(rows.at[pl.ds(0, sz)], out_sp.at[pl.ds(row0 + off, sz)])
        pltpu.sync_copy(wbuf.at[pl.ds(0, sz)], den_sp.at[pl.ds(row0 + off, sz)])
        off += sz
    plsc.subcore_barrier()

    ebase = worker * (K * C)

    @pl.loop(0, K)
    def _(k):
        base = ebase + k * C
        pltpu.sync_copy(src_hbm.at[pl.ds(base, C)], srcb.at[0])
        pltpu.sync_copy(dst_hbm.at[pl.ds(base, C)], dstb.at[0])
        for j in range(C // 16):
            sv = srcb[0, pl.ds(16 * j, 16)]
            dv = dstb[0, pl.ds(16 * j, 16)]
            e = plsc.load_gather(asrc_v, [sv]) + plsc.load_gather(adst_v, [dv])
            e = jnp.where(e < 0, e * jnp.float32(0.2), e)
            wbuf[pl.ds(16 * j, 16)] = jnp.exp(e - shift_vec)
        pltpu.sync_copy(h_hbm.at[srcb.at[0]], rows)

        @pl.loop(0, C)
        def _(r):
            wv = plsc.load_gather(wbuf, [jnp.full((16,), r, jnp.int32)])
            for t in range(H // 16):
                rows[r, pl.ds(16 * t, 16)] = rows[r, pl.ds(16 * t, 16)] * wv

        pltpu.sync_copy(rows, out_sp.at[dstb.at[0]], add=True)
        pltpu.sync_copy(wbuf, den_sp.at[dstb.at[0]], add=True)

    plsc.subcore_barrier()
    off = 0
    for sz in _ZCHUNKS:
        pltpu.sync_copy(out_sp.at[pl.ds(row0 + off, sz)],
                        out_hbm.at[cid, pl.ds(row0 + off, sz)])
        pltpu.sync_copy(den_sp.at[pl.ds(row0 + off, sz)],
                        den_hbm.at[cid, pl.ds(row0 + off, sz)])
        off += sz


def _edge_pass(h_pad, a_src, a_dst, shift, src_all, dst_all):
    mesh = plsc.VectorSubcoreMesh(core_axis_name="c", subcore_axis_name="s")
    kern = pl.kernel(
        _edge_body,
        out_type=[
            jax.ShapeDtypeStruct((NC, NP, H), jnp.float32),
            jax.ShapeDtypeStruct((NC, NP), jnp.float32),
        ],
        mesh=mesh,
        compiler_params=pltpu.CompilerParams(needs_layout_passes=False),
        scratch_types=[
            pltpu.VMEM((NP,), jnp.float32),
            pltpu.VMEM((NP,), jnp.float32),
            pltpu.VMEM((16,), jnp.float32),
            pltpu.VMEM((1, C), jnp.int32),
            pltpu.VMEM((1, C), jnp.int32),
            pltpu.VMEM((C,), jnp.float32),
            pltpu.VMEM((C, H), jnp.float32),
            pltpu.VMEM_SHARED((NP, H), jnp.float32),
            pltpu.VMEM_SHARED((NP,), jnp.float32),
        ],
    )
    return kern(h_pad, a_src, a_dst, shift, src_all, dst_all)


def _final_body(p_ref, d_ref, b_ref, g_ref, be_ref, m_ref, v_ref, o_ref):
    s = p_ref[0] + p_ref[1]
    den = d_ref[0] + d_ref[1] + 1e-16
    out = s / den + b_ref[...]
    out = jnp.maximum(out, 0.0)
    scale = g_ref[...] * jax.lax.rsqrt(v_ref[...] + 1e-5)
    o_ref[...] = (out - m_ref[...]) * scale + be_ref[...]


def _finalize(partials, denoms, bias, bn_gamma, bn_beta, bn_mean, bn_var):
    blk = 1000
    vec = lambda a: a.reshape(1, H)
    return pl.pallas_call(
        _final_body,
        grid=(N // blk,),
        in_specs=[
            pl.BlockSpec((NC, blk, H), lambda i: (0, i, 0)),
            pl.BlockSpec((NC, blk, 1), lambda i: (0, i, 0)),
            pl.BlockSpec((1, H), lambda i: (0, 0)),
            pl.BlockSpec((1, H), lambda i: (0, 0)),
            pl.BlockSpec((1, H), lambda i: (0, 0)),
            pl.BlockSpec((1, H), lambda i: (0, 0)),
            pl.BlockSpec((1, H), lambda i: (0, 0)),
        ],
        out_specs=pl.BlockSpec((blk, H), lambda i: (i, 0)),
        out_shape=jax.ShapeDtypeStruct((N, H), jnp.float32),
    )(partials, denoms.reshape(NC, NP, 1), vec(bias), vec(bn_gamma),
      vec(bn_beta), vec(bn_mean), vec(bn_var))


def kernel(x, edge_index, W, att_src, att_dst, bias, bn_gamma, bn_beta,
           bn_mean, bn_var):
    x_pad = jnp.pad(x, ((0, NP - N), (0, 0)))
    loop = jnp.arange(N, dtype=jnp.int32)
    pad = N + (jnp.arange(E_PAD - E_SELF, dtype=jnp.int32) % (NP - N))
    src_all = jnp.concatenate([edge_index[0], loop, pad])
    dst_all = jnp.concatenate([edge_index[1], loop, pad])

    h_pad, a_src, a_dst, shift = _project(x_pad, W, att_src, att_dst)
    partials, denoms = _edge_pass(
        h_pad, a_src.reshape(NP), a_dst.reshape(NP), shift.reshape(16),
        src_all, dst_all)
    return _finalize(partials, denoms, bias, bn_gamma, bn_beta, bn_mean,
                     bn_var)
